# Initial kernel scaffold; baseline (speedup 1.0000x reference)
import jax
import jax.numpy as jnp
from jax.experimental import pallas as pl


def kernel(x, edge_attr, node_W, node_b, edge_W, edge_b, msg_W1, msg_b1, msg_W2, msg_b2, upd_W1, upd_b1, upd_W2, upd_b2, ln_g, ln_b, edge_index):
    raise NotImplementedError("write your pallas kernel here")


# speedup vs baseline: 3.6835x; 3.6835x over previous
import functools

import jax
import jax.numpy as jnp
from jax import lax
from jax.experimental import pallas as pl
from jax.experimental.pallas import tpu as pltpu
from jax.experimental.pallas import tpu_sc as plsc

F32 = jnp.float32

_NC = 2
_NS = 16
_LN = 16
_NW = _NC * _NS


def _matmul_bias_body(x_ref, w_ref, b_ref, o_ref):
    o_ref[...] = (
        jnp.dot(x_ref[...], w_ref[...], preferred_element_type=F32) + b_ref[...]
    )


def _tc_matmul_bias(x, w, b, nb):
    n, _ = x.shape
    h = w.shape[1]
    return pl.pallas_call(
        _matmul_bias_body,
        grid=(n // nb,),
        in_specs=[
            pl.BlockSpec((nb, x.shape[1]), lambda i: (i, 0)),
            pl.BlockSpec(w.shape, lambda i: (0, 0)),
            pl.BlockSpec((1, h), lambda i: (0, 0)),
        ],
        out_specs=pl.BlockSpec((nb, h), lambda i: (i, 0)),
        out_shape=jax.ShapeDtypeStruct((n, h), F32),
    )(x, w, b.reshape(1, h))


def _ec_body(ea_ref, we_ref, ce_ref, o0_ref, o1_ref, o2_ref):
    ea = ea_ref[...]
    outs = (o0_ref, o1_ref, o2_ref)
    for l in range(3):
        outs[l][...] = (
            jnp.dot(ea, we_ref[l], preferred_element_type=F32) + ce_ref[l]
        )


def _tc_ec(edge_attr, we, ce, eb):
    e, ed = edge_attr.shape
    nl, _, h = we.shape
    return pl.pallas_call(
        _ec_body,
        grid=(e // eb,),
        in_specs=[
            pl.BlockSpec((eb, ed), lambda i: (i, 0)),
            pl.BlockSpec((nl, ed, h), lambda i: (0, 0, 0)),
            pl.BlockSpec((nl, 1, h), lambda i: (0, 0, 0)),
        ],
        out_specs=[pl.BlockSpec((eb, h), lambda i: (i, 0))] * 3,
        out_shape=[jax.ShapeDtypeStruct((e, h), F32)] * 3,
    )(edge_attr, we, ce.reshape(nl, 1, h))


def _ab_body(h_ref, wi_ref, wj_ref, a_ref, b_ref):
    hb = h_ref[...]
    a_ref[...] = jnp.dot(hb, wi_ref[...], preferred_element_type=F32)
    b_ref[...] = jnp.dot(hb, wj_ref[...], preferred_element_type=F32)


def _tc_ab(h, wi, wj, nb):
    n, d = h.shape
    return pl.pallas_call(
        _ab_body,
        grid=(n // nb,),
        in_specs=[
            pl.BlockSpec((nb, d), lambda i: (i, 0)),
            pl.BlockSpec(wi.shape, lambda i: (0, 0)),
            pl.BlockSpec(wj.shape, lambda i: (0, 0)),
        ],
        out_specs=[pl.BlockSpec((nb, d), lambda i: (i, 0))] * 2,
        out_shape=[jax.ShapeDtypeStruct((n, d), F32)] * 2,
    )(h, wi, wj)


def _upd_body(rp_ref, h_ref, deg_ref, w2_ref, b2_ref, u1a_ref, u1b_ref,
              ub1_ref, u2_ref, ub2_ref, g_ref, lb_ref, o_ref):
    rsum = rp_ref[0] + rp_ref[1]
    aggr = (
        jnp.dot(rsum, w2_ref[...], preferred_element_type=F32)
        + deg_ref[...] * b2_ref[...]
    )
    hb = h_ref[...]
    t = jnp.maximum(
        jnp.dot(hb, u1a_ref[...], preferred_element_type=F32)
        + jnp.dot(aggr, u1b_ref[...], preferred_element_type=F32)
        + ub1_ref[...],
        0.0,
    )
    v = hb + jnp.dot(t, u2_ref[...], preferred_element_type=F32) + ub2_ref[...]
    mu = jnp.mean(v, axis=1, keepdims=True)
    var = jnp.mean((v - mu) ** 2, axis=1, keepdims=True)
    o_ref[...] = (v - mu) * lax.rsqrt(var + 1e-5) * g_ref[...] + lb_ref[...]


def _tc_update(rp, h, deg2, w2, b2, u1a, u1b, ub1, u2, ub2, g, lb, nb):
    n, d = h.shape
    wspec = pl.BlockSpec((d, d), lambda i: (0, 0))
    vspec = pl.BlockSpec((1, d), lambda i: (0, 0))
    return pl.pallas_call(
        _upd_body,
        grid=(n // nb,),
        in_specs=[
            pl.BlockSpec((2, nb, d), lambda i: (0, i, 0)),
            pl.BlockSpec((nb, d), lambda i: (i, 0)),
            pl.BlockSpec((nb, 1), lambda i: (i, 0)),
            wspec, vspec, wspec, wspec, vspec, wspec, vspec, vspec, vspec,
        ],
        out_specs=pl.BlockSpec((nb, d), lambda i: (i, 0)),
        out_shape=jax.ShapeDtypeStruct((n, d), F32),
    )(rp, h, deg2, w2, b2.reshape(1, d), u1a, u1b, ub1.reshape(1, d),
      u2, ub2.reshape(1, d), g.reshape(1, d), lb.reshape(1, d))


@functools.lru_cache(maxsize=None)
def _make_edge_kernel(n, e, h):
    epw = e // _NW
    ch = 80
    nchunk = epw // ch
    zr = 32
    np_ = -(-n // (_NS * zr)) * (_NS * zr)
    rpt = np_ // _NS
    mesh = plsc.VectorSubcoreMesh(core_axis_name="c", subcore_axis_name="s")

    @functools.partial(
        pl.kernel,
        out_type=jax.ShapeDtypeStruct((_NC, np_, h), F32),
        mesh=mesh,
        scratch_types=[
            pltpu.VMEM_SHARED((np_, h), F32),
            pltpu.VMEM((ch,), jnp.int32),
            pltpu.VMEM((ch,), jnp.int32),
            pltpu.VMEM((ch, h), F32),
            pltpu.VMEM((ch, h), F32),
            pltpu.VMEM((ch, h), F32),
            pltpu.VMEM((ch, h), F32),
            pltpu.VMEM((32, h), F32),
            pltpu.SemaphoreType.DMA,
            pltpu.SemaphoreType.DMA,
            pltpu.SemaphoreType.DMA,
        ],
    )
    def edge_kernel(a_hbm, b_hbm, ec_hbm, src_hbm, dst_hbm, out_hbm,
                    acc_sh, idxd_v, idxs_v, a_v, b_v, e_v, m_v, z_v,
                    sem_a, sem_b, sem_s):
        cid = lax.axis_index("c")
        sid = lax.axis_index("s")
        wid = cid * _NS + sid
        base_e = wid * epw
        zero16 = jnp.zeros((_LN,), F32)

        def zrow(r, c_):
            for cc in range(h // _LN):
                z_v[r, pl.ds(cc * _LN, _LN)] = zero16
            return c_

        lax.fori_loop(0, zr, zrow, 0)
        for k in range(rpt // zr):
            pltpu.sync_copy(z_v, acc_sh.at[pl.ds(sid * rpt + k * zr, zr)])
        plsc.subcore_barrier()

        def chunk(i, c_):
            off = base_e + i * ch
            pltpu.sync_copy(dst_hbm.at[pl.ds(off, ch)], idxd_v)
            cp_a = pltpu.async_copy(a_hbm.at[idxd_v], a_v, sem_a)
            pltpu.sync_copy(src_hbm.at[pl.ds(off, ch)], idxs_v)
            cp_b = pltpu.async_copy(b_hbm.at[idxs_v], b_v, sem_b)
            pltpu.sync_copy(ec_hbm.at[pl.ds(off, ch)], e_v)
            cp_a.wait()
            cp_b.wait()

            def row(r, c2_):
                for cc in range(h // _LN):
                    s = pl.ds(cc * _LN, _LN)
                    m_v[r, s] = jnp.maximum(a_v[r, s] + b_v[r, s] + e_v[r, s],
                                            0.0)
                return c2_

            lax.fori_loop(0, ch, row, 0)
            pltpu.async_copy(m_v, acc_sh.at[idxd_v], sem_s, add=True).wait()
            return c_

        lax.fori_loop(0, nchunk, chunk, 0)
        plsc.subcore_barrier()
        rows = pl.ds(sid * rpt, rpt)
        pltpu.sync_copy(acc_sh.at[rows], out_hbm.at[cid, rows])

    return edge_kernel


@functools.lru_cache(maxsize=None)
def _make_deg_kernel(n, e):
    epw = e // _NW
    ch = 80
    nchunk = epw // ch
    zr = 32
    np_ = -(-n // (_NS * zr)) * (_NS * zr)
    rpt = np_ // _NS
    w = 128
    mesh = plsc.VectorSubcoreMesh(core_axis_name="c", subcore_axis_name="s")

    @functools.partial(
        pl.kernel,
        out_type=jax.ShapeDtypeStruct((_NC, np_, w), F32),
        mesh=mesh,
        scratch_types=[
            pltpu.VMEM_SHARED((np_, w), F32),
            pltpu.VMEM((ch,), jnp.int32),
            pltpu.VMEM((ch, w), F32),
            pltpu.VMEM((zr, w), F32),
            pltpu.SemaphoreType.DMA,
        ],
    )
    def deg_kernel(dst_hbm, out_hbm, acc_sh, idxd_v, one_v, z_v, sem_s):
        cid = lax.axis_index("c")
        sid = lax.axis_index("s")
        wid = cid * _NS + sid
        base_e = wid * epw
        zero16 = jnp.zeros((_LN,), F32)
        one16 = jnp.ones((_LN,), F32)

        def zrow(r, c_):
            for cc in range(w // _LN):
                z_v[r, pl.ds(cc * _LN, _LN)] = zero16
            return c_

        lax.fori_loop(0, zr, zrow, 0)
        for k in range(rpt // zr):
            pltpu.sync_copy(z_v, acc_sh.at[pl.ds(sid * rpt + k * zr, zr)])

        def orow(r, c_):
            for cc in range(w // _LN):
                one_v[r, pl.ds(cc * _LN, _LN)] = one16
            return c_

        lax.fori_loop(0, ch, orow, 0)
        plsc.subcore_barrier()

        def chunk(i, c_):
            pltpu.sync_copy(dst_hbm.at[pl.ds(base_e + i * ch, ch)], idxd_v)
            pltpu.async_copy(one_v, acc_sh.at[idxd_v], sem_s, add=True).wait()
            return c_

        lax.fori_loop(0, nchunk, chunk, 0)
        plsc.subcore_barrier()
        rows = pl.ds(sid * rpt, rpt)
        pltpu.sync_copy(acc_sh.at[rows], out_hbm.at[cid, rows])

    return deg_kernel


def kernel(x, edge_attr, node_W, node_b, edge_W, edge_b,
           msg_W1, msg_b1, msg_W2, msg_b2,
           upd_W1, upd_b1, upd_W2, upd_b2,
           ln_g, ln_b, edge_index):
    n, d = x.shape
    e = edge_index.shape[1]
    h = node_W.shape[1]
    nl = msg_W1.shape[0]
    nb = 400
    eb = 2000

    src = edge_index[0]
    dst = edge_index[1]

    w1i = msg_W1[:, :h, :]
    w1j = msg_W1[:, h:2 * h, :]
    w1e = msg_W1[:, 2 * h:, :]
    we = jnp.einsum('dh,lhk->ldk', edge_W, w1e)
    ce = jnp.einsum('h,lhk->lk', edge_b, w1e) + msg_b1

    h0 = _tc_matmul_bias(x, node_W, node_b, nb)
    ecs = _tc_ec(edge_attr, we, ce, eb)

    degp = _make_deg_kernel(n, e)(dst)
    deg2 = degp[0, :n, :1] + degp[1, :n, :1]

    edge_k = _make_edge_kernel(n, e, h)
    hcur = h0
    for l in range(nl):
        a, b = _tc_ab(hcur, w1i[l], w1j[l], nb)
        rp = edge_k(a, b, ecs[l], src, dst)
        hcur = _tc_update(
            rp, hcur, deg2, msg_W2[l], msg_b2[l],
            upd_W1[l, :h, :], upd_W1[l, h:, :], upd_b1[l],
            upd_W2[l], upd_b2[l], ln_g[l], ln_b[l], nb)
    return hcur
